# baseline (speedup 1.0000x reference)
import functools

import jax
import jax.numpy as jnp
from jax import lax
from jax.experimental import pallas as pl
from jax.experimental.pallas import tpu as pltpu
from jax.experimental.pallas import tpu_sc as plsc

H = 1024
E = 64
K = 2
F = 512
T = 2048
B = 128
MAXB = 96
NPAD = MAXB * B
CH = 512
NC = 2
NS = 16
NW = NC * NS
L = 16


def _router_body(x_ref, wg_ref, pos1_ref, pos2_ref, w1_ref, w2_ref,
                 be_ref, bv_ref, fe_ref, sl_ref):
    x = x_ref[0]
    logits = jnp.dot(x, wg_ref[...], preferred_element_type=jnp.float32)
    lanes = lax.broadcasted_iota(jnp.int32, (T, E), 1)
    m1 = jnp.max(logits, axis=1, keepdims=True)
    e1 = jnp.min(jnp.where(logits == m1, lanes, E), axis=1, keepdims=True)
    masked = jnp.where(lanes == e1, -jnp.inf, logits)
    m2 = jnp.max(masked, axis=1, keepdims=True)
    e2 = jnp.min(jnp.where(masked == m2, lanes, E), axis=1, keepdims=True)
    d = jnp.exp(m2 - m1)
    w1f = jnp.sum(1.0 / (1.0 + d), axis=1)
    w1_ref[...] = w1f
    w2_ref[...] = 1.0 - w1f

    oh1 = (lanes == e1).astype(jnp.float32)
    oh2 = (lanes == e2).astype(jnp.float32)

    r = lax.broadcasted_iota(jnp.int32, (CH, CH), 0)
    c = lax.broadcasted_iota(jnp.int32, (CH, CH), 1)
    tri = (r >= c).astype(jnp.float32)
    prefix = jnp.zeros((1, E), jnp.float32)
    cums, chunks = [], []
    for half in (oh1, oh2):
        for j in range(T // CH):
            blk = half[j * CH:(j + 1) * CH, :]
            loc = jnp.dot(tri, blk, preferred_element_type=jnp.float32) + prefix
            prefix = loc[CH - 1:CH, :]
            cums.append(loc)
            chunks.append(blk)
    counts = prefix
    blocks_e = jnp.ceil(counts / B)
    re = lax.broadcasted_iota(jnp.int32, (E, E), 0)
    ce = lax.broadcasted_iota(jnp.int32, (E, E), 1)
    tri_strict = (re < ce).astype(jnp.float32)
    poffset = jnp.dot(blocks_e, tri_strict,
                      preferred_element_type=jnp.float32) * B
    total = jnp.sum(blocks_e, keepdims=False) * B

    for idx, (cum, blk) in enumerate(zip(cums, chunks)):
        pos = (jnp.sum((cum + poffset) * blk, axis=1) - 1.0
               ).astype(jnp.int32)
        tgt = pos1_ref if idx < (T // CH) else pos2_ref
        row = (idx % (T // CH)) * CH
        tgt[row:row + CH] = pos

    bgrid = lax.broadcasted_iota(jnp.int32, (MAXB, E), 0).astype(jnp.float32) * B
    lanes_b = lax.broadcasted_iota(jnp.int32, (MAXB, E), 1)
    qual = (poffset <= bgrid) & (blocks_e > 0)
    be_col = jnp.max(jnp.where(qual, lanes_b, -1), axis=1, keepdims=True)
    be_ref[...] = jnp.max(jnp.where(qual, lanes_b, -1), axis=1)
    bv_ref[...] = jnp.max((bgrid < total).astype(jnp.int32), axis=1)

    be_prev = jnp.concatenate([jnp.full((1, 1), -7, jnp.int32),
                               be_col[:MAXB - 1, :]], axis=0)
    fetch = (be_col != be_prev).astype(jnp.float32)
    rb = lax.broadcasted_iota(jnp.int32, (MAXB, MAXB), 0)
    cb = lax.broadcasted_iota(jnp.int32, (MAXB, MAXB), 1)
    tri_b = (rb >= cb).astype(jnp.float32)
    nfetch = jnp.dot(tri_b, fetch, preferred_element_type=jnp.float32)
    slot = lax.rem(nfetch.astype(jnp.int32) - 1, 4)
    fe_ref[...] = jnp.sum(fetch, axis=1).astype(jnp.int32)
    sl_ref[...] = jnp.sum(slot, axis=1)


def _router(x, wg):
    return pl.pallas_call(
        _router_body,
        out_shape=(
            jax.ShapeDtypeStruct((T,), jnp.int32),
            jax.ShapeDtypeStruct((T,), jnp.int32),
            jax.ShapeDtypeStruct((T,), jnp.float32),
            jax.ShapeDtypeStruct((T,), jnp.float32),
            jax.ShapeDtypeStruct((MAXB,), jnp.int32),
            jax.ShapeDtypeStruct((MAXB,), jnp.int32),
            jax.ShapeDtypeStruct((MAXB,), jnp.int32),
            jax.ShapeDtypeStruct((MAXB,), jnp.int32),
        ),
    )(x, wg)


_RPW = NPAD // NW


def _dispatch_body(pos1_hbm, pos2_hbm, w1_hbm, w2_hbm, tok_hbm, wrow_hbm,
                   tok_loc, wrow_loc, posbuf, wbuf):
    cid = lax.axis_index("c")
    sid = lax.axis_index("s")
    wid = sid * NC + cid
    lo = wid * _RPW

    def zero_body(i, carry):
        tok_loc[pl.ds(i * L, L)] = jnp.zeros((L,), jnp.int32)
        wrow_loc[pl.ds(i * L, L)] = jnp.zeros((L,), jnp.float32)
        return carry
    lax.fori_loop(0, _RPW // L, zero_body, 0)
    for p_hbm, wv_hbm in ((pos1_hbm, w1_hbm), (pos2_hbm, w2_hbm)):
        pltpu.sync_copy(p_hbm, posbuf)
        pltpu.sync_copy(wv_hbm, wbuf)

        def sc_body(i, carry):
            pv = posbuf[pl.ds(i * L, L)] - lo
            wv = wbuf[pl.ds(i * L, L)]
            tv = lax.iota(jnp.int32, L) + i * L
            m = (pv >= 0) & (pv < _RPW)
            plsc.store_scatter(tok_loc, [pv], tv, mask=m)
            plsc.store_scatter(wrow_loc, [pv], wv, mask=m)
            return carry
        lax.fori_loop(0, T // L, sc_body, 0, unroll=2)
    pltpu.sync_copy(tok_loc, tok_hbm.at[pl.ds(lo, _RPW)])
    pltpu.sync_copy(wrow_loc, wrow_hbm.at[pl.ds(lo, _RPW)])


def _dispatch(pos1, pos2, w1, w2):
    return pl.kernel(
        _dispatch_body,
        out_type=(
            jax.ShapeDtypeStruct((NPAD,), jnp.int32),
            jax.ShapeDtypeStruct((NPAD,), jnp.float32),
        ),
        mesh=plsc.VectorSubcoreMesh(core_axis_name="c", subcore_axis_name="s"),
        compiler_params=pltpu.CompilerParams(needs_layout_passes=False),
        scratch_types=[
            pltpu.VMEM((_RPW,), jnp.int32),
            pltpu.VMEM((_RPW,), jnp.float32),
            pltpu.VMEM((T,), jnp.int32),
            pltpu.VMEM((T,), jnp.float32),
        ],
    )(pos1, pos2, w1, w2)


def _ffn_body(be_ref, bv_ref, fe_ref, sl_ref, x_ref, tok_ref, wr_ref,
              w1_hbm, w3_hbm, w2_hbm, y_ref,
              w1b, w3b, w2b, s1, s3, s2):
    i = pl.program_id(0)

    def issue(j):
        e = be_ref[j]
        s = sl_ref[j]
        pltpu.async_copy(w1_hbm.at[e], w1b.at[s], s1.at[s])
        pltpu.async_copy(w3_hbm.at[e], w3b.at[s], s3.at[s])
        pltpu.async_copy(w2_hbm.at[e], w2b.at[s], s2.at[s])

    @pl.when(i == 0)
    def _():
        for j in range(3):
            @pl.when(fe_ref[j] != 0)
            def _():
                issue(j)

    @pl.when(i + 3 < MAXB)
    def _():
        @pl.when(fe_ref[i + 3] != 0)
        def _():
            issue(i + 3)

    @pl.when(fe_ref[i] != 0)
    def _():
        e = be_ref[i]
        s = sl_ref[i]
        pltpu.make_async_copy(w1_hbm.at[e], w1b.at[s], s1.at[s]).wait()
        pltpu.make_async_copy(w3_hbm.at[e], w3b.at[s], s3.at[s]).wait()
        pltpu.make_async_copy(w2_hbm.at[e], w2b.at[s], s2.at[s]).wait()

    @pl.when(bv_ref[i] != 0)
    def _():
        s = sl_ref[i]
        sub = pl.ds(lax.rem(i, 8), 1)
        row = tok_ref[sub, :]
        toks = lax.broadcasted_iota(jnp.int32, (T, B), 0)
        selT = (toks == row).astype(jnp.float32)
        xs = lax.dot_general(selT, x_ref[0], (((0,), (0,)), ((), ())),
                             preferred_element_type=jnp.float32)
        a = jnp.dot(xs, w1b[s], preferred_element_type=jnp.float32)
        g = jnp.dot(xs, w3b[s], preferred_element_type=jnp.float32)
        h = a * jax.nn.sigmoid(a) * g
        y = jnp.dot(h, w2b[s], preferred_element_type=jnp.float32)
        wcol = wr_ref[sub, :].reshape(B, 1)
        y_ref[...] = y * wcol


def _ffn(be, bv, fe, sl, x, tok, wrow, w1, w3, w2):
    grid_spec = pltpu.PrefetchScalarGridSpec(
        num_scalar_prefetch=4,
        grid=(MAXB,),
        in_specs=[
            pl.BlockSpec((1, T, H), lambda i, be, bv, fe, sl: (0, 0, 0)),
            pl.BlockSpec((8, B), lambda i, be, bv, fe, sl: (i // 8, 0)),
            pl.BlockSpec((8, B), lambda i, be, bv, fe, sl: (i // 8, 0)),
            pl.BlockSpec(memory_space=pl.ANY),
            pl.BlockSpec(memory_space=pl.ANY),
            pl.BlockSpec(memory_space=pl.ANY),
        ],
        out_specs=pl.BlockSpec(
            (B, H),
            lambda i, be, bv, fe, sl: (jnp.where(bv[i] != 0, i, MAXB - 1), 0)),
        scratch_shapes=[
            pltpu.VMEM((4, H, F), jnp.float32),
            pltpu.VMEM((4, H, F), jnp.float32),
            pltpu.VMEM((4, F, H), jnp.float32),
            pltpu.SemaphoreType.DMA((4,)),
            pltpu.SemaphoreType.DMA((4,)),
            pltpu.SemaphoreType.DMA((4,)),
        ],
    )
    return pl.pallas_call(
        _ffn_body,
        grid_spec=grid_spec,
        out_shape=jax.ShapeDtypeStruct((NPAD, H), jnp.float32),
    )(be, bv, fe, sl, x, tok, wrow, w1, w3, w2)


_CCH = 16


def _combine_body(pos1_hbm, pos2_hbm, y_hbm, out_hbm,
                  i1a, i2a, i1b, i2b, r1a, r2a, r1b, r2b,
                  g1a, g2a, g1b, g2b, wsa, wsb):
    cid = lax.axis_index("c")
    sid = lax.axis_index("s")
    wid = sid * NC + cid
    tok_per_w = T // NW
    nch = tok_per_w // _CCH
    i1 = (i1a, i1b)
    i2 = (i2a, i2b)
    r1 = (r1a, r1b)
    r2 = (r2a, r2b)
    g1 = (g1a, g1b)
    g2 = (g2a, g2b)
    ws = (wsa, wsb)
    cps = {}
    wrs = {}

    def fire(j):
        b = j % 2
        cb = wid * tok_per_w + j * _CCH
        pltpu.sync_copy(pos1_hbm.at[pl.ds(cb, _CCH)], i1[b])
        pltpu.sync_copy(pos2_hbm.at[pl.ds(cb, _CCH)], i2[b])
        cps[j] = (pltpu.async_copy(y_hbm.at[i1[b]], r1[b], g1[b]),
                  pltpu.async_copy(y_hbm.at[i2[b]], r2[b], g2[b]))

    fire(0)
    for j in range(nch):
        b = j % 2
        if j + 1 < nch:
            if j >= 1:
                wrs[j - 1].wait()
            fire(j + 1)
        cps[j][0].wait()
        cps[j][1].wait()

        def add_body(k, carry):
            row = k // (H // L)
            col = (k % (H // L)) * L
            r1[b][row, pl.ds(col, L)] = (r1[b][row, pl.ds(col, L)] +
                                         r2[b][row, pl.ds(col, L)])
            return carry
        lax.fori_loop(0, _CCH * (H // L), add_body, 0, unroll=8)
        cb = wid * tok_per_w + j * _CCH
        wrs[j] = pltpu.async_copy(r1[b], out_hbm.at[0, pl.ds(cb, _CCH)], ws[b])
    wrs[nch - 2].wait()
    wrs[nch - 1].wait()


def _combine(pos1, pos2, y):
    return pl.kernel(
        _combine_body,
        out_type=jax.ShapeDtypeStruct((1, T, H), jnp.float32),
        mesh=plsc.VectorSubcoreMesh(core_axis_name="c", subcore_axis_name="s"),
        scratch_types=[
            pltpu.VMEM((_CCH,), jnp.int32),
            pltpu.VMEM((_CCH,), jnp.int32),
            pltpu.VMEM((_CCH,), jnp.int32),
            pltpu.VMEM((_CCH,), jnp.int32),
            pltpu.VMEM((_CCH, H), jnp.float32),
            pltpu.VMEM((_CCH, H), jnp.float32),
            pltpu.VMEM((_CCH, H), jnp.float32),
            pltpu.VMEM((_CCH, H), jnp.float32),
            pltpu.SemaphoreType.DMA,
            pltpu.SemaphoreType.DMA,
            pltpu.SemaphoreType.DMA,
            pltpu.SemaphoreType.DMA,
            pltpu.SemaphoreType.DMA,
            pltpu.SemaphoreType.DMA,
        ],
    )(pos1, pos2, y)


def kernel(hidden_states, Wg, W1, W3, W2):
    pos1, pos2, w1, w2, be, bv, fe, sl = _router(hidden_states, Wg)
    tok, wrow = _dispatch(pos1, pos2, w1, w2)
    y = _ffn(be, bv, fe, sl, hidden_states,
             tok.reshape(MAXB, B), wrow.reshape(MAXB, B), W1, W3, W2)
    return _combine(pos1, pos2, y)


# speedup vs baseline: 1.0083x; 1.0083x over previous
import functools

import jax
import jax.numpy as jnp
from jax import lax
from jax.experimental import pallas as pl
from jax.experimental.pallas import tpu as pltpu
from jax.experimental.pallas import tpu_sc as plsc

H = 1024
E = 64
K = 2
F = 512
T = 2048
B = 128
MAXB = 96
NPAD = MAXB * B
CH = 512
NC = 2
NS = 16
NW = NC * NS
L = 16


def _router_body(x_ref, wg_ref, pos1_ref, pos2_ref, w1_ref, w2_ref,
                 be_ref, bv_ref, fe_ref, sl_ref):
    x = x_ref[0]
    logits = jnp.dot(x, wg_ref[...], preferred_element_type=jnp.float32)
    lanes = lax.broadcasted_iota(jnp.int32, (T, E), 1)
    m1 = jnp.max(logits, axis=1, keepdims=True)
    e1 = jnp.min(jnp.where(logits == m1, lanes, E), axis=1, keepdims=True)
    masked = jnp.where(lanes == e1, -jnp.inf, logits)
    m2 = jnp.max(masked, axis=1, keepdims=True)
    e2 = jnp.min(jnp.where(masked == m2, lanes, E), axis=1, keepdims=True)
    d = jnp.exp(m2 - m1)
    w1f = jnp.sum(1.0 / (1.0 + d), axis=1)
    w1_ref[...] = w1f
    w2_ref[...] = 1.0 - w1f

    oh1 = (lanes == e1).astype(jnp.float32)
    oh2 = (lanes == e2).astype(jnp.float32)

    r = lax.broadcasted_iota(jnp.int32, (CH, CH), 0)
    c = lax.broadcasted_iota(jnp.int32, (CH, CH), 1)
    tri = (r >= c).astype(jnp.float32)
    prefix = jnp.zeros((1, E), jnp.float32)
    cums, chunks = [], []
    for half in (oh1, oh2):
        for j in range(T // CH):
            blk = half[j * CH:(j + 1) * CH, :]
            loc = jnp.dot(tri, blk, preferred_element_type=jnp.float32) + prefix
            prefix = loc[CH - 1:CH, :]
            cums.append(loc)
            chunks.append(blk)
    counts = prefix
    blocks_e = jnp.ceil(counts / B)
    re = lax.broadcasted_iota(jnp.int32, (E, E), 0)
    ce = lax.broadcasted_iota(jnp.int32, (E, E), 1)
    tri_strict = (re < ce).astype(jnp.float32)
    poffset = jnp.dot(blocks_e, tri_strict,
                      preferred_element_type=jnp.float32) * B
    total = jnp.sum(blocks_e, keepdims=False) * B

    for idx, (cum, blk) in enumerate(zip(cums, chunks)):
        pos = (jnp.sum((cum + poffset) * blk, axis=1) - 1.0
               ).astype(jnp.int32)
        tgt = pos1_ref if idx < (T // CH) else pos2_ref
        row = (idx % (T // CH)) * CH
        tgt[row:row + CH] = pos

    bgrid = lax.broadcasted_iota(jnp.int32, (MAXB, E), 0).astype(jnp.float32) * B
    lanes_b = lax.broadcasted_iota(jnp.int32, (MAXB, E), 1)
    qual = (poffset <= bgrid) & (blocks_e > 0)
    be_col = jnp.max(jnp.where(qual, lanes_b, -1), axis=1, keepdims=True)
    be_ref[...] = jnp.max(jnp.where(qual, lanes_b, -1), axis=1)
    bv_ref[...] = jnp.max((bgrid < total).astype(jnp.int32), axis=1)

    be_prev = jnp.concatenate([jnp.full((1, 1), -7, jnp.int32),
                               be_col[:MAXB - 1, :]], axis=0)
    fetch = (be_col != be_prev).astype(jnp.float32)
    rb = lax.broadcasted_iota(jnp.int32, (MAXB, MAXB), 0)
    cb = lax.broadcasted_iota(jnp.int32, (MAXB, MAXB), 1)
    tri_b = (rb >= cb).astype(jnp.float32)
    nfetch = jnp.dot(tri_b, fetch, preferred_element_type=jnp.float32)
    slot = lax.rem(nfetch.astype(jnp.int32) - 1, 3)
    fe_ref[...] = jnp.sum(fetch, axis=1).astype(jnp.int32)
    sl_ref[...] = jnp.sum(slot, axis=1)


def _router(x, wg):
    return pl.pallas_call(
        _router_body,
        out_shape=(
            jax.ShapeDtypeStruct((T,), jnp.int32),
            jax.ShapeDtypeStruct((T,), jnp.int32),
            jax.ShapeDtypeStruct((T,), jnp.float32),
            jax.ShapeDtypeStruct((T,), jnp.float32),
            jax.ShapeDtypeStruct((MAXB,), jnp.int32),
            jax.ShapeDtypeStruct((MAXB,), jnp.int32),
            jax.ShapeDtypeStruct((MAXB,), jnp.int32),
            jax.ShapeDtypeStruct((MAXB,), jnp.int32),
        ),
    )(x, wg)


_RPW = NPAD // NW


def _dispatch_body(pos1_hbm, pos2_hbm, w1_hbm, w2_hbm, tok_hbm, wrow_hbm,
                   tok_loc, wrow_loc, posbuf, wbuf):
    cid = lax.axis_index("c")
    sid = lax.axis_index("s")
    wid = sid * NC + cid
    lo = wid * _RPW

    def zero_body(i, carry):
        tok_loc[pl.ds(i * L, L)] = jnp.zeros((L,), jnp.int32)
        wrow_loc[pl.ds(i * L, L)] = jnp.zeros((L,), jnp.float32)
        return carry
    lax.fori_loop(0, _RPW // L, zero_body, 0)
    for p_hbm, wv_hbm in ((pos1_hbm, w1_hbm), (pos2_hbm, w2_hbm)):
        pltpu.sync_copy(p_hbm, posbuf)
        pltpu.sync_copy(wv_hbm, wbuf)

        def sc_body(i, carry):
            pv = posbuf[pl.ds(i * L, L)] - lo
            wv = wbuf[pl.ds(i * L, L)]
            tv = lax.iota(jnp.int32, L) + i * L
            m = (pv >= 0) & (pv < _RPW)
            plsc.store_scatter(tok_loc, [pv], tv, mask=m)
            plsc.store_scatter(wrow_loc, [pv], wv, mask=m)
            return carry
        lax.fori_loop(0, T // L, sc_body, 0, unroll=2)
    pltpu.sync_copy(tok_loc, tok_hbm.at[pl.ds(lo, _RPW)])
    pltpu.sync_copy(wrow_loc, wrow_hbm.at[pl.ds(lo, _RPW)])


def _dispatch(pos1, pos2, w1, w2):
    return pl.kernel(
        _dispatch_body,
        out_type=(
            jax.ShapeDtypeStruct((NPAD,), jnp.int32),
            jax.ShapeDtypeStruct((NPAD,), jnp.float32),
        ),
        mesh=plsc.VectorSubcoreMesh(core_axis_name="c", subcore_axis_name="s"),
        compiler_params=pltpu.CompilerParams(needs_layout_passes=False),
        scratch_types=[
            pltpu.VMEM((_RPW,), jnp.int32),
            pltpu.VMEM((_RPW,), jnp.float32),
            pltpu.VMEM((T,), jnp.int32),
            pltpu.VMEM((T,), jnp.float32),
        ],
    )(pos1, pos2, w1, w2)


def _ffn_body(be_ref, bv_ref, fe_ref, sl_ref, x_ref, tok_ref, wr_ref,
              w1_hbm, w3_hbm, w2_hbm, y_ref,
              w1b, w3b, w2b, s1, s3, s2):
    i = pl.program_id(0)

    def issue(j):
        e = be_ref[j]
        s = sl_ref[j]
        pltpu.async_copy(w1_hbm.at[e], w1b.at[s], s1.at[s])
        pltpu.async_copy(w3_hbm.at[e], w3b.at[s], s3.at[s])
        pltpu.async_copy(w2_hbm.at[e], w2b.at[s], s2.at[s])

    @pl.when(i == 0)
    def _():
        for j in range(2):
            @pl.when(fe_ref[j] != 0)
            def _():
                issue(j)

    @pl.when(i + 2 < MAXB)
    def _():
        @pl.when(fe_ref[i + 2] != 0)
        def _():
            issue(i + 2)

    @pl.when(fe_ref[i] != 0)
    def _():
        e = be_ref[i]
        s = sl_ref[i]
        pltpu.make_async_copy(w1_hbm.at[e], w1b.at[s], s1.at[s]).wait()
        pltpu.make_async_copy(w3_hbm.at[e], w3b.at[s], s3.at[s]).wait()
        pltpu.make_async_copy(w2_hbm.at[e], w2b.at[s], s2.at[s]).wait()

    @pl.when(bv_ref[i] != 0)
    def _():
        s = sl_ref[i]
        sub = pl.ds(lax.rem(i, 8), 1)
        row = tok_ref[sub, :]
        toks = lax.broadcasted_iota(jnp.int32, (T, B), 0)
        selT = (toks == row).astype(jnp.float32)
        xs = lax.dot_general(selT, x_ref[0], (((0,), (0,)), ((), ())),
                             preferred_element_type=jnp.float32)
        a = jnp.dot(xs, w1b[s], preferred_element_type=jnp.float32)
        g = jnp.dot(xs, w3b[s], preferred_element_type=jnp.float32)
        h = a * jax.nn.sigmoid(a) * g
        y = jnp.dot(h, w2b[s], preferred_element_type=jnp.float32)
        wcol = wr_ref[sub, :].reshape(B, 1)
        y_ref[...] = y * wcol


def _ffn(be, bv, fe, sl, x, tok, wrow, w1, w3, w2):
    grid_spec = pltpu.PrefetchScalarGridSpec(
        num_scalar_prefetch=4,
        grid=(MAXB,),
        in_specs=[
            pl.BlockSpec((1, T, H), lambda i, be, bv, fe, sl: (0, 0, 0)),
            pl.BlockSpec((8, B), lambda i, be, bv, fe, sl: (i // 8, 0)),
            pl.BlockSpec((8, B), lambda i, be, bv, fe, sl: (i // 8, 0)),
            pl.BlockSpec(memory_space=pl.ANY),
            pl.BlockSpec(memory_space=pl.ANY),
            pl.BlockSpec(memory_space=pl.ANY),
        ],
        out_specs=pl.BlockSpec(
            (B, H),
            lambda i, be, bv, fe, sl: (jnp.where(bv[i] != 0, i, MAXB - 1), 0)),
        scratch_shapes=[
            pltpu.VMEM((3, H, F), jnp.float32),
            pltpu.VMEM((3, H, F), jnp.float32),
            pltpu.VMEM((3, F, H), jnp.float32),
            pltpu.SemaphoreType.DMA((3,)),
            pltpu.SemaphoreType.DMA((3,)),
            pltpu.SemaphoreType.DMA((3,)),
        ],
    )
    return pl.pallas_call(
        _ffn_body,
        grid_spec=grid_spec,
        out_shape=jax.ShapeDtypeStruct((NPAD, H), jnp.float32),
    )(be, bv, fe, sl, x, tok, wrow, w1, w3, w2)


_CCH = 16


def _combine_body(pos1_hbm, pos2_hbm, y_hbm, out_hbm,
                  i1a, i2a, i1b, i2b, r1a, r2a, r1b, r2b,
                  g1a, g2a, g1b, g2b, wsa, wsb):
    cid = lax.axis_index("c")
    sid = lax.axis_index("s")
    wid = sid * NC + cid
    tok_per_w = T // NW
    nch = tok_per_w // _CCH
    i1 = (i1a, i1b)
    i2 = (i2a, i2b)
    r1 = (r1a, r1b)
    r2 = (r2a, r2b)
    g1 = (g1a, g1b)
    g2 = (g2a, g2b)
    ws = (wsa, wsb)
    cps = {}
    wrs = {}

    def fire(j):
        b = j % 2
        cb = wid * tok_per_w + j * _CCH
        pltpu.sync_copy(pos1_hbm.at[pl.ds(cb, _CCH)], i1[b])
        pltpu.sync_copy(pos2_hbm.at[pl.ds(cb, _CCH)], i2[b])
        cps[j] = (pltpu.async_copy(y_hbm.at[i1[b]], r1[b], g1[b]),
                  pltpu.async_copy(y_hbm.at[i2[b]], r2[b], g2[b]))

    fire(0)
    for j in range(nch):
        b = j % 2
        if j + 1 < nch:
            if j >= 1:
                wrs[j - 1].wait()
            fire(j + 1)
        cps[j][0].wait()
        cps[j][1].wait()

        def add_body(k, carry):
            row = k // (H // L)
            col = (k % (H // L)) * L
            r1[b][row, pl.ds(col, L)] = (r1[b][row, pl.ds(col, L)] +
                                         r2[b][row, pl.ds(col, L)])
            return carry
        lax.fori_loop(0, _CCH * (H // L), add_body, 0, unroll=8)
        cb = wid * tok_per_w + j * _CCH
        wrs[j] = pltpu.async_copy(r1[b], out_hbm.at[0, pl.ds(cb, _CCH)], ws[b])
    wrs[nch - 2].wait()
    wrs[nch - 1].wait()


def _combine(pos1, pos2, y):
    return pl.kernel(
        _combine_body,
        out_type=jax.ShapeDtypeStruct((1, T, H), jnp.float32),
        mesh=plsc.VectorSubcoreMesh(core_axis_name="c", subcore_axis_name="s"),
        scratch_types=[
            pltpu.VMEM((_CCH,), jnp.int32),
            pltpu.VMEM((_CCH,), jnp.int32),
            pltpu.VMEM((_CCH,), jnp.int32),
            pltpu.VMEM((_CCH,), jnp.int32),
            pltpu.VMEM((_CCH, H), jnp.float32),
            pltpu.VMEM((_CCH, H), jnp.float32),
            pltpu.VMEM((_CCH, H), jnp.float32),
            pltpu.VMEM((_CCH, H), jnp.float32),
            pltpu.SemaphoreType.DMA,
            pltpu.SemaphoreType.DMA,
            pltpu.SemaphoreType.DMA,
            pltpu.SemaphoreType.DMA,
            pltpu.SemaphoreType.DMA,
            pltpu.SemaphoreType.DMA,
        ],
    )(pos1, pos2, y)


def kernel(hidden_states, Wg, W1, W3, W2):
    pos1, pos2, w1, w2, be, bv, fe, sl = _router(hidden_states, Wg)
    tok, wrow = _dispatch(pos1, pos2, w1, w2)
    y = _ffn(be, bv, fe, sl, hidden_states,
             tok.reshape(MAXB, B), wrow.reshape(MAXB, B), W1, W3, W2)
    return _combine(pos1, pos2, y)
